# Initial kernel scaffold; baseline (speedup 1.0000x reference)
import jax
import jax.numpy as jnp
from jax.experimental import pallas as pl


def kernel(x, edge_index, batch, c1_W1, c1_b1, c1_W2, c1_b2, c2_W1, c2_b1, c2_W2, c2_b2, c3_W1, c3_b1, c3_W2, c3_b2, c4_W1, c4_b1, c4_W2, c4_b2, head_W, head_b):
    raise NotImplementedError("write your pallas kernel here")


# speedup vs baseline: 6.8861x; 6.8861x over previous
import functools

import jax
import jax.numpy as jnp
from jax import lax
from jax.experimental import pallas as pl
from jax.experimental.pallas import tpu as pltpu
from jax.experimental.pallas import tpu_sc as plsc

N = 10000
K = 4
E = N * K
NP = 10240
EP = NP * 4
RB = 256
CB = 256
NRB = NP // RB
BIGB = 1 << 30
BIGI = jnp.iinfo(jnp.int32).max
NW = 32
CHUNK = 128
F32 = jnp.float32
I32 = jnp.int32


def _sc_mesh():
    return plsc.VectorSubcoreMesh(core_axis_name="c", subcore_axis_name="s")


def _take16(x, idx):
    dn = lax.GatherDimensionNumbers(offset_dims=(), collapsed_slice_dims=(0,),
                                    start_index_map=(0,))
    return lax.gather(x, idx.reshape(16, 1), dn, (1,),
                      mode=lax.GatherScatterMode.PROMISE_IN_BOUNDS)


def _sc_gather_pair(table_a, idx_a, table_b, idx_b):
    per_w = EP // NW
    nch = per_w // CHUNK

    @functools.partial(
        pl.kernel, mesh=_sc_mesh(),
        out_type=(jax.ShapeDtypeStruct((EP, 128), F32),
                  jax.ShapeDtypeStruct((EP, 128), F32)),
        scratch_types=[pltpu.VMEM((CHUNK,), I32),
                       pltpu.VMEM((CHUNK, 128), F32),
                       pltpu.SemaphoreType.DMA],
    )
    def kfn(ta, tb, ia, ib, oa, ob, idx_v, rows_v, sem):
        wid = lax.axis_index("s") * 2 + lax.axis_index("c")
        base0 = wid * per_w
        for tab, iref, oref in ((ta, ia, oa), (tb, ib, ob)):
            for ch in range(nch):
                b = base0 + ch * CHUNK
                pltpu.sync_copy(iref.at[pl.ds(b, CHUNK)], idx_v)
                pltpu.async_copy(tab.at[idx_v], rows_v, sem).wait()
                pltpu.sync_copy(rows_v, oref.at[pl.ds(b, CHUNK)])

    return kfn(table_a, table_b, idx_a, idx_b)


def _sc_gather_rows(table, idx):
    per_w = EP // NW
    nch = per_w // CHUNK

    @functools.partial(
        pl.kernel, mesh=_sc_mesh(),
        out_type=jax.ShapeDtypeStruct((EP, 256), F32),
        scratch_types=[pltpu.VMEM((CHUNK,), I32),
                       pltpu.VMEM((CHUNK, 256), F32),
                       pltpu.SemaphoreType.DMA],
    )
    def kfn(tab, iref, oref, idx_v, rows_v, sem):
        wid = lax.axis_index("s") * 2 + lax.axis_index("c")
        base0 = wid * per_w
        for ch in range(nch):
            b = base0 + ch * CHUNK
            pltpu.sync_copy(iref.at[pl.ds(b, CHUNK)], idx_v)
            pltpu.async_copy(tab.at[idx_v], rows_v, sem).wait()
            pltpu.sync_copy(rows_v, oref.at[pl.ds(b, CHUNK)])

    return kfn(table, idx)


def _tc_segment_sum(dst2d, msgs):
    EB = 512

    def body(dst_ref, m_ref, o_ref):
        e = pl.program_id(0)

        @pl.when(e == 0)
        def _():
            o_ref[...] = jnp.zeros((NP, 256), F32)

        def step(i, _):
            d = dst_ref[0, 0, i]
            o_ref[pl.ds(d, 1), :] += m_ref[pl.ds(i, 1), :]
            return 0

        lax.fori_loop(0, EB, step, 0)

    return pl.pallas_call(
        body,
        grid=(EP // EB,),
        in_specs=[
            pl.BlockSpec((1, 1, EB), lambda e: (e, 0, 0),
                         memory_space=pltpu.SMEM),
            pl.BlockSpec((EB, 256), lambda e: (e, 0)),
        ],
        out_specs=pl.BlockSpec((NP, 256), lambda e: (0, 0)),
        out_shape=jax.ShapeDtypeStruct((NP, 256), F32),
    )(dst2d, msgs)


def _edge_mlp1_full(xi, xj, w1cat, b1, w2, b2):
    EB = 512

    def body(xi_ref, xj_ref, w1_ref, b1_ref, w2_ref, b2_ref, o_ref):
        a = xi_ref[...]
        feat = jnp.concatenate([a, xj_ref[...] - a], axis=1)
        m = jnp.dot(feat, w1_ref[...], preferred_element_type=F32)
        m = jnp.maximum(m + b1_ref[...], 0.0)
        m = jnp.dot(m, w2_ref[...], preferred_element_type=F32)
        o_ref[...] = jnp.maximum(m + b2_ref[...], 0.0)

    return pl.pallas_call(
        body,
        grid=(EP // EB,),
        in_specs=[
            pl.BlockSpec((EB, 128), lambda e: (e, 0)),
            pl.BlockSpec((EB, 128), lambda e: (e, 0)),
            pl.BlockSpec((256, 128), lambda e: (0, 0)),
            pl.BlockSpec((1, 128), lambda e: (0, 0)),
            pl.BlockSpec((128, 256), lambda e: (0, 0)),
            pl.BlockSpec((1, 256), lambda e: (0, 0)),
        ],
        out_specs=pl.BlockSpec((EB, 256), lambda e: (e, 0)),
        out_shape=jax.ShapeDtypeStruct((EP, 256), F32),
    )(xi, xj, w1cat, b1, w2, b2)


def _edge_mlp_full(h, g3, w1p, b1p, w2p, b2):
    NB = 128

    def body(h_ref, g_ref, w1_ref, b1_ref, w2_ref, b2_ref, o_ref):
        hd = h_ref[...]
        acc = None
        for j in range(4):
            feat = jnp.concatenate([hd, g_ref[:, j, :] - hd], axis=1)
            m = jnp.dot(feat, w1_ref[...], preferred_element_type=F32)
            m = jnp.maximum(m + b1_ref[...], 0.0)
            m = jnp.dot(m, w2_ref[...], preferred_element_type=F32)
            m = jnp.maximum(m + b2_ref[...], 0.0)
            acc = m if acc is None else acc + m
        o_ref[...] = acc

    return pl.pallas_call(
        body,
        grid=(NP // NB,),
        in_specs=[
            pl.BlockSpec((NB, 256), lambda i: (i, 0)),
            pl.BlockSpec((NB, 4, 256), lambda i: (i, 0, 0)),
            pl.BlockSpec((512, 384), lambda i: (0, 0)),
            pl.BlockSpec((1, 384), lambda i: (0, 0)),
            pl.BlockSpec((384, 256), lambda i: (0, 0)),
            pl.BlockSpec((1, 256), lambda i: (0, 0)),
        ],
        out_specs=pl.BlockSpec((NB, 256), lambda i: (i, 0)),
        out_shape=jax.ShapeDtypeStruct((NP, 256), F32),
    )(h, g3, w1p, b1p, w2p, b2)


def _knn(h, ht3, bt_row, bt_col3, cb_lo, cb_hi):
    INF = float("inf")

    def body(lo_ref, hi_ref, h_ref, ht_ref, br_ref, bc_ref, o_ref):
        r = pl.program_id(0)
        hr = h_ref[...]
        sqr = jnp.sum(hr * hr, axis=1, keepdims=True)
        brow = br_ref[...]
        rgid = lax.broadcasted_iota(I32, (RB, 1), 0) + r * RB

        def col_step(cb, carry):
            vals, idxs = carry
            htc = ht_ref[pl.ds(cb, 1)]
            htc = htc.reshape(256, CB)
            dot = jnp.dot(hr, htc, preferred_element_type=F32)
            sqc = jnp.sum(htc * htc, axis=0, keepdims=True)
            d = (sqr + sqc) - 2.0 * dot
            bcol = bc_ref[pl.ds(cb, 1)].reshape(1, CB)
            cgid = lax.broadcasted_iota(I32, (1, CB), 1) + cb * CB
            bad = (brow != bcol) | (rgid == cgid)
            d = jnp.where(bad, INF, d)
            bv, bi = [], []
            for _ in range(4):
                m = jnp.min(d, axis=1, keepdims=True)
                am = jnp.min(jnp.where(d == m, cgid, BIGI), axis=1,
                             keepdims=True)
                d = jnp.where((d == m) & (cgid == am), INF, d)
                bv.append(m)
                bi.append(am)
            cv = jnp.concatenate([vals] + bv, axis=1)
            ci = jnp.concatenate([idxs] + bi, axis=1)
            nv, ni = [], []
            for _ in range(4):
                m = jnp.min(cv, axis=1, keepdims=True)
                am = jnp.min(jnp.where(cv == m, ci, BIGI), axis=1,
                             keepdims=True)
                cv = jnp.where((cv == m) & (ci == am), INF, cv)
                nv.append(m)
                ni.append(am)
            return (jnp.concatenate(nv, axis=1), jnp.concatenate(ni, axis=1))

        init = (jnp.full((RB, 4), INF, F32), jnp.zeros((RB, 4), I32))
        vals, idxs = lax.fori_loop(lo_ref[r], hi_ref[r], col_step, init)
        idxs = jnp.where(vals == INF, 0, idxs)
        o_ref[...] = jnp.concatenate([idxs, jnp.zeros((RB, 4), I32)], axis=1)

    return pl.pallas_call(
        body,
        grid=(NRB,),
        in_specs=[
            pl.BlockSpec(memory_space=pltpu.SMEM),
            pl.BlockSpec(memory_space=pltpu.SMEM),
            pl.BlockSpec((RB, 256), lambda r: (r, 0)),
            pl.BlockSpec((NRB, 256, CB), lambda r: (0, 0, 0)),
            pl.BlockSpec((RB, 1), lambda r: (r, 0)),
            pl.BlockSpec((NRB, 1, CB), lambda r: (0, 0, 0)),
        ],
        out_specs=pl.BlockSpec((RB, 8), lambda r: (r, 0)),
        out_shape=jax.ShapeDtypeStruct((NP, 8), I32),
    )(cb_lo, cb_hi, h, ht3, bt_row, bt_col3)


def _head(xp, h1, h2, h3, h4, wx, w1, w2, w3, w4, bh):
    NB = 256

    def body(x_ref, a_ref, b_ref, c_ref, d_ref, wx_ref, w1_ref, w2_ref,
             w3_ref, w4_ref, bh_ref, o_ref):
        o = jnp.dot(x_ref[...], wx_ref[...], preferred_element_type=F32)
        o = o + jnp.dot(a_ref[...], w1_ref[...], preferred_element_type=F32)
        o = o + jnp.dot(b_ref[...], w2_ref[...], preferred_element_type=F32)
        o = o + jnp.dot(c_ref[...], w3_ref[...], preferred_element_type=F32)
        o = o + jnp.dot(d_ref[...], w4_ref[...], preferred_element_type=F32)
        o_ref[...] = o + bh_ref[...]

    return pl.pallas_call(
        body,
        grid=(NP // NB,),
        in_specs=[
            pl.BlockSpec((NB, 128), lambda i: (i, 0)),
            pl.BlockSpec((NB, 256), lambda i: (i, 0)),
            pl.BlockSpec((NB, 256), lambda i: (i, 0)),
            pl.BlockSpec((NB, 256), lambda i: (i, 0)),
            pl.BlockSpec((NB, 256), lambda i: (i, 0)),
            pl.BlockSpec((128, 8), lambda i: (0, 0)),
            pl.BlockSpec((256, 8), lambda i: (0, 0)),
            pl.BlockSpec((256, 8), lambda i: (0, 0)),
            pl.BlockSpec((256, 8), lambda i: (0, 0)),
            pl.BlockSpec((256, 8), lambda i: (0, 0)),
            pl.BlockSpec((1, 8), lambda i: (0, 0)),
        ],
        out_specs=pl.BlockSpec((NB, 8), lambda i: (i, 0)),
        out_shape=jax.ShapeDtypeStruct((NP, 8), F32),
    )(xp, h1, h2, h3, h4, wx, w1, w2, w3, w4, bh)


def kernel(x, edge_index, batch,
           c1_W1, c1_b1, c1_W2, c1_b2,
           c2_W1, c2_b1, c2_W2, c2_b2,
           c3_W1, c3_b1, c3_W2, c3_b2,
           c4_W1, c4_b1, c4_W2, c4_b2,
           head_W, head_b):
    x = x.astype(F32)
    ei = edge_index.astype(I32)
    bt = batch.astype(I32)

    x128 = jnp.zeros((NP, 128), F32).at[:N, :5].set(x)
    bt_pad = jnp.concatenate([bt, jnp.full((NP - N,), BIGB, I32)])
    bt_row = bt_pad.reshape(NP, 1)
    bt_col3 = bt_pad.reshape(NRB, 1, CB)
    firsts = bt_pad[:: RB]
    lasts = bt_pad[RB - 1 :: RB]
    lo_el = jnp.searchsorted(bt_pad, firsts, side="left").astype(I32)
    hi_el = jnp.searchsorted(bt_pad, lasts, side="right").astype(I32)
    cb_lo = lo_el // CB
    cb_hi = (hi_el + CB - 1) // CB

    pad_e = jnp.zeros((EP - E,), I32)
    src1 = jnp.concatenate([ei[0], pad_e])
    dst1g = jnp.concatenate([ei[1], pad_e])
    dst1s = jnp.concatenate([ei[1], jnp.full((EP - E,), N + 200, I32)])
    xi, xj = _sc_gather_pair(x128, dst1g, x128, src1)
    w1cat = (jnp.zeros((256, 128), F32)
             .at[0:5].set(c1_W1[0:5])
             .at[128:133].set(c1_W1[5:10]))
    msgs = _edge_mlp1_full(xi, xj, w1cat, c1_b1.reshape(1, 128),
                           c1_W2, c1_b2.reshape(1, 256))
    h = _tc_segment_sum(dst1s.reshape(EP // 512, 1, 512), msgs)

    hs = []
    for (W1, b1, W2, b2) in ((c2_W1, c2_b1, c2_W2, c2_b2),
                             (c3_W1, c3_b1, c3_W2, c3_b2),
                             (c4_W1, c4_b1, c4_W2, c4_b2)):
        hs.append(h)
        ht3 = h.reshape(NRB, CB, 256).transpose(0, 2, 1)
        idx8 = _knn(h, ht3, bt_row, bt_col3, cb_lo, cb_hi)
        src = idx8[:, :4].reshape(EP)
        g = _sc_gather_rows(h, src)
        w1p = jnp.zeros((512, 384), F32).at[:, :336].set(W1)
        b1p = jnp.zeros((1, 384), F32).at[0, :336].set(b1)
        w2p = jnp.zeros((384, 256), F32).at[:336].set(W2)
        h = _edge_mlp_full(h, g.reshape(NP, 4, 256), w1p, b1p,
                           w2p, b2.reshape(1, 256))
    hs.append(h)

    wx = jnp.zeros((128, 8), F32).at[:5, :2].set(head_W[0:5])
    whs = [jnp.zeros((256, 8), F32).at[:, :2].set(head_W[5 + 256 * l:5 + 256 * (l + 1)])
           for l in range(4)]
    bh = jnp.zeros((1, 8), F32).at[0, :2].set(head_b)
    out = _head(x128, hs[0], hs[1], hs[2], hs[3], wx, *whs, bh)
    return out[:N, :2]
